# Initial kernel scaffold; baseline (speedup 1.0000x reference)
import jax
import jax.numpy as jnp
from jax.experimental import pallas as pl


def kernel(h, input_ids, attention_mask, is_operator, token_digit_value, token_digits_full):
    raise NotImplementedError("write your pallas kernel here")


# speedup vs baseline: 7.2682x; 7.2682x over previous
import functools

import jax
import jax.numpy as jnp
from jax import lax
from jax.experimental import pallas as pl
from jax.experimental.pallas import tpu as pltpu
from jax.experimental.pallas import tpu_sc as plsc

NC = 2
NS = 16
L = 16
NW = NC * NS
RPW = 32
KP = 16
MW = 3136


def _sc_extract(ids_t, B, S, opbits, tdf_pad):
    mesh = plsc.VectorSubcoreMesh(core_axis_name="c", subcore_axis_name="s")

    @functools.partial(
        pl.kernel,
        out_type=[
            jax.ShapeDtypeStruct((B, KP), jnp.float32),
            jax.ShapeDtypeStruct((B, KP), jnp.float32),
        ],
        mesh=mesh,
        compiler_params=pltpu.CompilerParams(
            needs_layout_passes=False, use_tc_tiling_on_sc=False),
        scratch_types=[
            pltpu.VMEM((RPW * S,), jnp.int32),
            pltpu.VMEM((MW,), jnp.int32),
            pltpu.VMEM((RPW, KP), jnp.float32),
            pltpu.VMEM((RPW, KP), jnp.float32),
            pltpu.VMEM((RPW,), jnp.int32),
            pltpu.VMEM((RPW,), jnp.int32),
            pltpu.VMEM((RPW,), jnp.int32),
            pltpu.VMEM((RPW,), jnp.int32),
            pltpu.VMEM((RPW,), jnp.int32),
            pltpu.SemaphoreType.DMA,
        ],
    )
    def sc_kernel(ids_hbm, opbits_hbm, tdf_hbm, da_hbm, db_hbm,
                  ids_v, mask_v, da_v, db_v,
                  st_min, st_a, st_b, st_prev, st_pend, sem):
        wid = lax.axis_index("s") * NC + lax.axis_index("c")
        base = wid * RPW
        pltpu.sync_copy(ids_hbm.at[pl.ds(wid * S * RPW, S * RPW)], ids_v)
        pltpu.sync_copy(opbits_hbm, mask_v)
        for g in range(RPW // L):
            off = g * L
            sl = pl.ds(off, L)
            ids0 = ids_v[sl]
            ids1 = ids_v[pl.ds(RPW + off, L)]
            st_min[sl] = jnp.full((L,), S, jnp.int32)
            st_a[sl] = ids0
            st_b[sl] = ids0
            st_prev[sl] = ids0
            st_pend[sl] = jnp.zeros((L,), jnp.int32)

            def body(s, c):
                cur = ids_v[pl.ds(s * RPW + off, L)]
                word = plsc.load_gather(
                    mask_v, [lax.shift_right_logical(cur, 5)])
                bit = lax.shift_right_logical(word, cur & 31) & 1
                minpos = st_min[sl]
                st_b[sl] = jnp.where(st_pend[sl] != 0, cur, st_b[sl])
                newly = (bit != 0) & (minpos >= S)
                st_min[sl] = jnp.where(
                    newly, jnp.zeros((L,), jnp.int32) + s, minpos)
                st_a[sl] = jnp.where(newly, st_prev[sl], st_a[sl])
                st_pend[sl] = newly.astype(jnp.int32)
                st_prev[sl] = cur
                return c

            lax.fori_loop(0, S, body, 0)
            a_ids = st_a[sl]
            b_ids = jnp.where(st_pend[sl] != 0, st_prev[sl], st_b[sl])
            nf = st_min[sl] >= S
            a_ids = jnp.where(nf, ids0, a_ids)
            b_ids = jnp.where(nf, ids1, b_ids)
            pltpu.async_copy(tdf_hbm.at[a_ids], da_v.at[sl], sem).wait()
            pltpu.async_copy(tdf_hbm.at[b_ids], db_v.at[sl], sem).wait()
        pltpu.sync_copy(da_v, da_hbm.at[pl.ds(base, RPW)])
        pltpu.sync_copy(db_v, db_hbm.at[pl.ds(base, RPW)])

    return sc_kernel(ids_t, opbits, tdf_pad)


def _tc_broadcast(da_flat, db_flat, rep, S, K):
    B = da_flat.shape[0]
    BB = 128
    SK = S * K

    def body(da_ref, db_ref, rep_ref, oa_ref, ob_ref):
        m = rep_ref[...]
        oa_ref[...] = jnp.dot(da_ref[...], m,
                              preferred_element_type=jnp.float32)
        ob_ref[...] = jnp.dot(db_ref[...], m,
                              preferred_element_type=jnp.float32)

    return pl.pallas_call(
        body,
        grid=(B // BB,),
        in_specs=[
            pl.BlockSpec((BB, KP), lambda i: (i, 0)),
            pl.BlockSpec((BB, KP), lambda i: (i, 0)),
            pl.BlockSpec((KP, SK), lambda i: (0, 0)),
        ],
        out_specs=[
            pl.BlockSpec((BB, SK), lambda i: (i, 0)),
            pl.BlockSpec((BB, SK), lambda i: (i, 0)),
        ],
        out_shape=[
            jax.ShapeDtypeStruct((B, SK), jnp.float32),
            jax.ShapeDtypeStruct((B, SK), jnp.float32),
        ],
    )(da_flat, db_flat, rep)


def kernel(h, input_ids, attention_mask, is_operator, token_digit_value,
           token_digits_full):
    B, S = input_ids.shape
    Vv, K = token_digits_full.shape
    ids_t = input_ids.reshape(NW, RPW, S).transpose(0, 2, 1).reshape(-1)
    ops = is_operator.astype(jnp.uint32).reshape(-1, 32)
    words = (ops << jnp.arange(32, dtype=jnp.uint32)[None, :]).sum(
        axis=1, dtype=jnp.uint32)
    opbits = lax.bitcast_convert_type(
        jnp.zeros((MW,), jnp.uint32).at[:words.shape[0]].set(words),
        jnp.int32)
    tdf_pad = jnp.pad(token_digits_full, ((0, 0), (0, KP - K)))
    rep = (jnp.arange(S * K, dtype=jnp.int32)[None, :] % K
           == jnp.arange(KP, dtype=jnp.int32)[:, None]).astype(jnp.float32)
    da_flat, db_flat = _sc_extract(ids_t, B, S, opbits, tdf_pad)
    oa, ob = _tc_broadcast(da_flat, db_flat, rep, S, K)
    d_a = oa.reshape(B, S, K)
    d_b = ob.reshape(B, S, K)
    return (d_a, d_b, d_a, d_b)
